# speedup vs baseline: 171.4348x; 171.4348x over previous
import jax
import jax.numpy as jnp
from jax import lax
from jax.experimental import pallas as pl
from jax.experimental.pallas import tpu as pltpu
from jax.experimental.pallas import tpu_sc as plsc

N = 10000
D = 128
NUM_GRAPHS = 64
NUM_ELEMENTS = 100
NUM_KEYS = NUM_GRAPHS * NUM_ELEMENTS
HALF_KEYS = NUM_KEYS // 2
NUM_TILES = 16
CHUNK = 128
ROWS_PER_TILE = 632
NCHUNK = 5
ACC_ROWS = 3328
ZERO_ROWS = ACC_ROWS // NUM_TILES
OUT_ROWS_PER_TILE = HALF_KEYS // NUM_TILES


def _sc_body(out_hbm, z_hbm, b_hbm, o_hbm, rowbuf, zbuf, bbuf, keybuf, acc):
    c = lax.axis_index("c")
    s = lax.axis_index("s")

    def _zero_row(i, carry):
        for j in range(D // 16):
            rowbuf[i, pl.ds(j * 16, 16)] = jnp.zeros((16,), jnp.float32)
        return carry

    lax.fori_loop(0, CHUNK, _zero_row, 0)
    zbase = s * ZERO_ROWS
    pltpu.sync_copy(rowbuf, acc.at[pl.ds(zbase, CHUNK)])
    pltpu.sync_copy(rowbuf.at[pl.ds(0, ZERO_ROWS - CHUNK)],
                    acc.at[pl.ds(zbase + CHUNK, ZERO_ROWS - CHUNK)])
    plsc.subcore_barrier()

    lo = s * ROWS_PER_TILE
    hi = jnp.minimum(lo + ROWS_PER_TILE, N)
    trash = HALF_KEYS + s
    key_base = c * HALF_KEYS

    for k in range(NCHUNK):
        start_l = lo + k * CHUNK
        a = jnp.minimum(start_l, N - CHUNK)
        hi_k = jnp.minimum(start_l + CHUNK, hi)
        pltpu.sync_copy(out_hbm.at[pl.ds(a, CHUNK)], rowbuf)
        pltpu.sync_copy(z_hbm.at[pl.ds(a, CHUNK)], zbuf)
        pltpu.sync_copy(b_hbm.at[pl.ds(a, CHUNK)], bbuf)
        for j in range(CHUNK // 16):
            zv = zbuf[pl.ds(j * 16, 16)]
            bv = bbuf[pl.ds(j * 16, 16)]
            grow = a + j * 16 + lax.iota(jnp.int32, 16)
            local = bv * NUM_ELEMENTS + zv - 1 - key_base
            valid = ((grow >= start_l) & (grow < hi_k)
                     & (local >= 0) & (local < HALF_KEYS))
            keybuf[pl.ds(j * 16, 16)] = jnp.where(valid, local, trash)
        pltpu.sync_copy(rowbuf, acc.at[keybuf], add=True)

    plsc.subcore_barrier()

    obase = s * OUT_ROWS_PER_TILE
    pltpu.sync_copy(acc.at[pl.ds(obase, CHUNK)],
                    o_hbm.at[pl.ds(key_base + obase, CHUNK)])
    rem = OUT_ROWS_PER_TILE - CHUNK
    pltpu.sync_copy(acc.at[pl.ds(obase + CHUNK, rem)],
                    o_hbm.at[pl.ds(key_base + obase + CHUNK, rem)])


@jax.jit
def _pool_call(out, z32, b32):
    mesh = plsc.VectorSubcoreMesh(core_axis_name="c", subcore_axis_name="s")
    return pl.kernel(
        _sc_body,
        out_type=jax.ShapeDtypeStruct((NUM_KEYS, D), jnp.float32),
        mesh=mesh,
        scratch_types=[
            pltpu.VMEM((CHUNK, D), jnp.float32),
            pltpu.VMEM((CHUNK,), jnp.int32),
            pltpu.VMEM((CHUNK,), jnp.int32),
            pltpu.VMEM((CHUNK,), jnp.int32),
            pltpu.VMEM_SHARED((ACC_ROWS, D), jnp.float32),
        ],
    )(out, z32, b32)


def kernel(out, z, batch):
    pooled = _pool_call(out, z.astype(jnp.int32), batch.astype(jnp.int32))
    return pooled.reshape(NUM_GRAPHS, NUM_ELEMENTS * D)
